# baseline (speedup 1.0000x reference)
import functools

import jax
import jax.numpy as jnp
from jax.experimental import pallas as pl
from jax.experimental.pallas import tpu as pltpu

S = 2048
D = 768
E = 8
K = 2
H = 3072
CAP = 512
TS = 512
NEG = -1e30


def _router_body(x_ref, noise_ref, wg_ref, bg_ref, wn_ref, bn_ref,
                 ws_ref, bs_ref, slot_ref, gate_ref, ns_ref):
    x = x_ref[...]
    logits = jnp.dot(x, wg_ref[...], preferred_element_type=jnp.float32) + bg_ref[...]
    nlog = jnp.dot(x, wn_ref[...], preferred_element_type=jnp.float32) + bn_ref[...]
    sp = jnp.maximum(nlog, 0.0) + jnp.log(1.0 + jnp.exp(-jnp.abs(nlog)))
    noisy = logits + noise_ref[...] * sp

    iota_e = jax.lax.broadcasted_iota(jnp.int32, (S, E), 1)
    m1 = jnp.max(noisy, axis=1, keepdims=True)
    e1 = jnp.min(jnp.where(noisy == m1, iota_e, E), axis=1, keepdims=True)
    masked = jnp.where(iota_e == e1, NEG, noisy)
    m2 = jnp.max(masked, axis=1, keepdims=True)
    e2 = jnp.min(jnp.where(masked == m2, iota_e, E), axis=1, keepdims=True)
    sel = (iota_e == e1) | (iota_e == e2)

    ex = jnp.where(sel, jnp.exp(noisy - m1), 0.0)
    gate_ref[...] = ex / jnp.sum(ex, axis=1, keepdims=True)

    slogit = jnp.dot(x, ws_ref[...], preferred_element_type=jnp.float32) + bs_ref[...]
    ns = (slogit <= 0.0).astype(jnp.float32)
    n_ns = jnp.sum(ns)
    cap = jnp.floor(n_ns * (K / E))

    m = jnp.where(sel, ns, 0.0)
    ci = jax.lax.broadcasted_iota(jnp.int32, (128, 128), 0)
    cj = jax.lax.broadcasted_iota(jnp.int32, (128, 128), 1)
    ltri = (ci >= cj).astype(jnp.float32)
    off = jnp.zeros((1, E), jnp.float32)
    ranks = []
    for c in range(S // 128):
        mc = m[c * 128:(c + 1) * 128, :]
        incl = jnp.dot(ltri, mc, preferred_element_type=jnp.float32) + off
        ranks.append(incl - 1.0)
        off = off + jnp.sum(mc, axis=0, keepdims=True)
    rank = jnp.concatenate(ranks, axis=0)
    keep = (m > 0.0) & (rank < cap)
    slot_ref[...] = jnp.where(keep, rank, -1.0)
    ns_ref[...] = ns


def _dispatch_body(ntt, slot_ref, gate_ref, x_ref, xd_ref, gd_ref,
                   xa_scr, ga_scr):
    e = pl.program_id(0)
    tt = pl.program_id(1)
    iota_e = jax.lax.broadcasted_iota(jnp.int32, (TS, E), 1)
    ecol = (iota_e == e).astype(jnp.float32)
    slot_col = jnp.sum(slot_ref[...] * ecol, axis=1, keepdims=True)
    gate_col = jnp.sum(gate_ref[...] * ecol, axis=1, keepdims=True)
    iota_s = jax.lax.broadcasted_iota(jnp.int32, (CAP, TS), 0)
    slot_i = slot_col.astype(jnp.int32).reshape(1, TS)
    pf = (slot_i == iota_s).astype(jnp.float32)
    xw = x_ref[...].astype(jnp.float32)
    xp = jnp.dot(pf, xw,
                 preferred_element_type=jnp.float32).astype(jnp.bfloat16)
    gp = jnp.dot(pf, gate_col, preferred_element_type=jnp.float32)

    @pl.when(tt == 0)
    def _init():
        xa_scr[...] = xp
        ga_scr[...] = gp

    @pl.when(tt != 0)
    def _acc():
        xa_scr[...] = xa_scr[...] + xp
        ga_scr[...] = ga_scr[...] + gp

    @pl.when(tt == ntt - 1)
    def _emit():
        xd_ref[0] = xa_scr[...]
        gd_ref[0] = ga_scr[...]


def _mlp_body(nhc, xd_ref, gd_ref, w1_ref, b1_ref, w2_ref, b2_ref,
              yo_ref, y_scr):
    hc = pl.program_id(1)
    h = jnp.maximum(
        jnp.dot(xd_ref[0], w1_ref[0], preferred_element_type=jnp.float32)
        + b1_ref[0], 0.0).astype(jnp.bfloat16)
    part = jnp.dot(h, w2_ref[0], preferred_element_type=jnp.float32)

    @pl.when(hc == 0)
    def _y0():
        y_scr[...] = part

    @pl.when(hc != 0)
    def _yn():
        y_scr[...] = y_scr[...] + part

    @pl.when(hc == nhc - 1)
    def _emit():
        yo_ref[0] = ((y_scr[...] + b2_ref[0]) * gd_ref[0]).astype(jnp.bfloat16)


def _combine_body(slot_ref, ns_ref, x_ref, yo_ref, out_ref, acc_scr):
    e = pl.program_id(1)
    iota_e = jax.lax.broadcasted_iota(jnp.int32, (TS, E), 1)
    ecol = (iota_e == e).astype(jnp.float32)
    slot_col = jnp.sum(slot_ref[...] * ecol, axis=1, keepdims=True)
    slot_i = slot_col.astype(jnp.int32)
    iota_c = jax.lax.broadcasted_iota(jnp.int32, (TS, CAP), 1)
    p2 = (slot_i == iota_c).astype(jnp.float32).astype(jnp.bfloat16)
    upd = jnp.dot(p2, yo_ref[0], preferred_element_type=jnp.float32)

    @pl.when(e == 0)
    def _init():
        acc_scr[...] = jnp.zeros_like(acc_scr)

    acc = acc_scr[...] + upd
    acc_scr[...] = acc

    @pl.when(e == E - 1)
    def _final():
        out_ref[...] = jnp.where(ns_ref[...] > 0.0, acc, x_ref[...])


def kernel(x, noise, Wg, bg, Wn, bn, Ws, bs, W1, b1, W2, b2):
    xf = x.reshape(S, D)
    xbf = xf.astype(jnp.bfloat16)
    nf = noise.reshape(S, E)

    slot, gate, ns = pl.pallas_call(
        _router_body,
        out_shape=(
            jax.ShapeDtypeStruct((S, E), jnp.float32),
            jax.ShapeDtypeStruct((S, E), jnp.float32),
            jax.ShapeDtypeStruct((S, 1), jnp.float32),
        ),
    )(xf, nf, Wg, bg.reshape(1, E), Wn, bn.reshape(1, E),
      Ws, bs.reshape(1, 1))

    ntt = S // TS
    xd, gd = pl.pallas_call(
        functools.partial(_dispatch_body, ntt),
        grid=(E, ntt),
        in_specs=[
            pl.BlockSpec((TS, E), lambda e, tt: (tt, 0)),
            pl.BlockSpec((TS, E), lambda e, tt: (tt, 0)),
            pl.BlockSpec((TS, D), lambda e, tt: (tt, 0)),
        ],
        out_specs=(
            pl.BlockSpec((1, CAP, D), lambda e, tt: (e, 0, 0)),
            pl.BlockSpec((1, CAP, 1), lambda e, tt: (e, 0, 0)),
        ),
        out_shape=(
            jax.ShapeDtypeStruct((E, CAP, D), jnp.bfloat16),
            jax.ShapeDtypeStruct((E, CAP, 1), jnp.float32),
        ),
        scratch_shapes=[
            pltpu.VMEM((CAP, D), jnp.bfloat16),
            pltpu.VMEM((CAP, 1), jnp.float32),
        ],
    )(slot, gate, xbf)

    hblk = 1536
    nhc = H // hblk
    yo = pl.pallas_call(
        functools.partial(_mlp_body, nhc),
        grid=(E, nhc),
        in_specs=[
            pl.BlockSpec((1, CAP, D), lambda e, hc: (e, 0, 0)),
            pl.BlockSpec((1, CAP, 1), lambda e, hc: (e, 0, 0)),
            pl.BlockSpec((1, D, hblk), lambda e, hc: (e, 0, hc)),
            pl.BlockSpec((1, 1, hblk), lambda e, hc: (e, 0, hc)),
            pl.BlockSpec((1, hblk, D), lambda e, hc: (e, hc, 0)),
            pl.BlockSpec((1, 1, D), lambda e, hc: (e, 0, 0)),
        ],
        out_specs=pl.BlockSpec((1, CAP, D), lambda e, hc: (e, 0, 0)),
        out_shape=jax.ShapeDtypeStruct((E, CAP, D), jnp.bfloat16),
        scratch_shapes=[pltpu.VMEM((CAP, D), jnp.float32)],
    )(xd, gd, W1.astype(jnp.bfloat16), b1.reshape(E, 1, H),
      W2.astype(jnp.bfloat16), b2.reshape(E, 1, D))

    ntt = S // TS
    out = pl.pallas_call(
        _combine_body,
        grid=(ntt, E),
        in_specs=[
            pl.BlockSpec((TS, E), lambda tt, e: (tt, 0)),
            pl.BlockSpec((TS, 1), lambda tt, e: (tt, 0)),
            pl.BlockSpec((TS, D), lambda tt, e: (tt, 0)),
            pl.BlockSpec((1, CAP, D), lambda tt, e: (e, 0, 0)),
        ],
        out_specs=pl.BlockSpec((TS, D), lambda tt, e: (tt, 0)),
        out_shape=jax.ShapeDtypeStruct((S, D), jnp.float32),
        scratch_shapes=[pltpu.VMEM((TS, D), jnp.float32)],
    )(slot, ns, xf, yo)

    return out.reshape(1, S, D)


# speedup vs baseline: 4.5151x; 4.5151x over previous
import functools

import jax
import jax.numpy as jnp
from jax.experimental import pallas as pl
from jax.experimental.pallas import tpu as pltpu

S = 2048
D = 768
E = 8
K = 2
H = 3072
CAP = 512
TS = 512
NEG = -1e30


def _router_body(x_ref, noise_ref, wg_ref, bg_ref, wn_ref, bn_ref,
                 ws_ref, bs_ref, slot_ref, gate_ref, ns_ref):
    x = x_ref[...]
    logits = jnp.dot(x, wg_ref[...], preferred_element_type=jnp.float32) + bg_ref[...]
    nlog = jnp.dot(x, wn_ref[...], preferred_element_type=jnp.float32) + bn_ref[...]
    sp = jnp.maximum(nlog, 0.0) + jnp.log(1.0 + jnp.exp(-jnp.abs(nlog)))
    noisy = logits + noise_ref[...] * sp

    iota_e = jax.lax.broadcasted_iota(jnp.int32, (S, E), 1)
    m1 = jnp.max(noisy, axis=1, keepdims=True)
    e1 = jnp.min(jnp.where(noisy == m1, iota_e, E), axis=1, keepdims=True)
    masked = jnp.where(iota_e == e1, NEG, noisy)
    m2 = jnp.max(masked, axis=1, keepdims=True)
    e2 = jnp.min(jnp.where(masked == m2, iota_e, E), axis=1, keepdims=True)
    sel = (iota_e == e1) | (iota_e == e2)

    ex = jnp.where(sel, jnp.exp(noisy - m1), 0.0)
    gate_ref[...] = ex / jnp.sum(ex, axis=1, keepdims=True)

    slogit = jnp.dot(x, ws_ref[...], preferred_element_type=jnp.float32) + bs_ref[...]
    ns = (slogit <= 0.0).astype(jnp.float32)
    n_ns = jnp.sum(ns)
    cap = jnp.floor(n_ns * (K / E))

    m = jnp.where(sel, ns, 0.0)
    ci = jax.lax.broadcasted_iota(jnp.int32, (128, 128), 0)
    cj = jax.lax.broadcasted_iota(jnp.int32, (128, 128), 1)
    ltri = (ci >= cj).astype(jnp.float32)
    off = jnp.zeros((1, E), jnp.float32)
    ranks = []
    for c in range(S // 128):
        mc = m[c * 128:(c + 1) * 128, :]
        incl = jnp.dot(ltri, mc, preferred_element_type=jnp.float32) + off
        ranks.append(incl - 1.0)
        off = off + jnp.sum(mc, axis=0, keepdims=True)
    rank = jnp.concatenate(ranks, axis=0)
    keep = (m > 0.0) & (rank < cap)
    slot_ref[...] = jnp.where(keep, rank, -1.0)
    ns_ref[...] = ns


def _dispatch_body(ntt, slot_ref, x_ref, xd_ref, xa_scr):
    e = pl.program_id(0)
    tt = pl.program_id(1)
    iota_e = jax.lax.broadcasted_iota(jnp.int32, (TS, E), 1)
    ecol = (iota_e == e).astype(jnp.float32)
    slot_col = jnp.sum(slot_ref[...] * ecol, axis=1, keepdims=True)
    slot_i = slot_col.astype(jnp.int32)
    iota_c = jax.lax.broadcasted_iota(jnp.int32, (TS, CAP), 1)
    pt = (slot_i == iota_c).astype(jnp.float32).astype(jnp.bfloat16)
    xp = jax.lax.dot_general(pt, x_ref[...], (((0,), (0,)), ((), ())),
                             preferred_element_type=jnp.float32
                             ).astype(jnp.bfloat16)

    @pl.when(tt == 0)
    def _init():
        xa_scr[...] = xp

    @pl.when(tt != 0)
    def _acc():
        xa_scr[...] = xa_scr[...] + xp

    @pl.when(tt == ntt - 1)
    def _emit():
        xd_ref[0] = xa_scr[...]


def _mlp_body(nhc, xd_ref, w1_ref, b1_ref, w2_ref, b2_ref,
              yo_ref, y_scr):
    hc = pl.program_id(1)
    h = jnp.maximum(
        jnp.dot(xd_ref[0], w1_ref[0], preferred_element_type=jnp.float32)
        + b1_ref[0], 0.0).astype(jnp.bfloat16)
    part = jnp.dot(h, w2_ref[0], preferred_element_type=jnp.float32)

    @pl.when(hc == 0)
    def _y0():
        y_scr[...] = part

    @pl.when(hc != 0)
    def _yn():
        y_scr[...] = y_scr[...] + part

    @pl.when(hc == nhc - 1)
    def _emit():
        yo_ref[0] = (y_scr[...] + b2_ref[0]).astype(jnp.bfloat16)


def _combine_body(slot_ref, gate_ref, ns_ref, x_ref, yo_ref, out_ref,
                  acc_scr):
    e = pl.program_id(1)
    iota_e = jax.lax.broadcasted_iota(jnp.int32, (TS, E), 1)
    ecol = (iota_e == e).astype(jnp.float32)
    slot_col = jnp.sum(slot_ref[...] * ecol, axis=1, keepdims=True)
    gate_col = jnp.sum(gate_ref[...] * ecol, axis=1, keepdims=True)
    slot_i = slot_col.astype(jnp.int32)
    iota_c = jax.lax.broadcasted_iota(jnp.int32, (TS, CAP), 1)
    p2 = jnp.where(slot_i == iota_c, gate_col, 0.0).astype(jnp.bfloat16)
    upd = jnp.dot(p2, yo_ref[0], preferred_element_type=jnp.float32)

    @pl.when(e == 0)
    def _init():
        acc_scr[...] = jnp.zeros_like(acc_scr)

    acc = acc_scr[...] + upd
    acc_scr[...] = acc

    @pl.when(e == E - 1)
    def _final():
        out_ref[...] = jnp.where(ns_ref[...] > 0.0, acc, x_ref[...])


def kernel(x, noise, Wg, bg, Wn, bn, Ws, bs, W1, b1, W2, b2):
    xf = x.reshape(S, D)
    xbf = xf.astype(jnp.bfloat16)
    nf = noise.reshape(S, E)

    slot, gate, ns = pl.pallas_call(
        _router_body,
        out_shape=(
            jax.ShapeDtypeStruct((S, E), jnp.float32),
            jax.ShapeDtypeStruct((S, E), jnp.float32),
            jax.ShapeDtypeStruct((S, 1), jnp.float32),
        ),
    )(xf, nf, Wg, bg.reshape(1, E), Wn, bn.reshape(1, E),
      Ws, bs.reshape(1, 1))

    ntt = S // TS
    xd = pl.pallas_call(
        functools.partial(_dispatch_body, ntt),
        grid=(E, ntt),
        in_specs=[
            pl.BlockSpec((TS, E), lambda e, tt: (tt, 0)),
            pl.BlockSpec((TS, D), lambda e, tt: (tt, 0)),
        ],
        out_specs=pl.BlockSpec((1, CAP, D), lambda e, tt: (e, 0, 0)),
        out_shape=jax.ShapeDtypeStruct((E, CAP, D), jnp.bfloat16),
        scratch_shapes=[
            pltpu.VMEM((CAP, D), jnp.bfloat16),
        ],
    )(slot, xbf)

    hblk = 1536
    nhc = H // hblk
    yo = pl.pallas_call(
        functools.partial(_mlp_body, nhc),
        grid=(E, nhc),
        in_specs=[
            pl.BlockSpec((1, CAP, D), lambda e, hc: (e, 0, 0)),
            pl.BlockSpec((1, D, hblk), lambda e, hc: (e, 0, hc)),
            pl.BlockSpec((1, 1, hblk), lambda e, hc: (e, 0, hc)),
            pl.BlockSpec((1, hblk, D), lambda e, hc: (e, hc, 0)),
            pl.BlockSpec((1, 1, D), lambda e, hc: (e, 0, 0)),
        ],
        out_specs=pl.BlockSpec((1, CAP, D), lambda e, hc: (e, 0, 0)),
        out_shape=jax.ShapeDtypeStruct((E, CAP, D), jnp.bfloat16),
        scratch_shapes=[pltpu.VMEM((CAP, D), jnp.float32)],
    )(xd, W1.astype(jnp.bfloat16), b1.reshape(E, 1, H),
      W2.astype(jnp.bfloat16), b2.reshape(E, 1, D))

    ntt = S // TS
    out = pl.pallas_call(
        _combine_body,
        grid=(ntt, E),
        in_specs=[
            pl.BlockSpec((TS, E), lambda tt, e: (tt, 0)),
            pl.BlockSpec((TS, E), lambda tt, e: (tt, 0)),
            pl.BlockSpec((TS, 1), lambda tt, e: (tt, 0)),
            pl.BlockSpec((TS, D), lambda tt, e: (tt, 0)),
            pl.BlockSpec((1, CAP, D), lambda tt, e: (e, 0, 0)),
        ],
        out_specs=pl.BlockSpec((TS, D), lambda tt, e: (tt, 0)),
        out_shape=jax.ShapeDtypeStruct((S, D), jnp.float32),
        scratch_shapes=[pltpu.VMEM((TS, D), jnp.float32)],
    )(slot, gate, ns, xf, yo)

    return out.reshape(1, S, D)
